# baseline (speedup 1.0000x reference)
import jax
import jax.numpy as jnp
from jax import lax
from jax.experimental import pallas as pl
from jax.experimental.pallas import tpu as pltpu
from jax.experimental.pallas import tpu_sc as plsc

EMBED = 128
NSQ = 64
NCOMB = NSQ * NSQ
LANES = 16
IDXW = 128
CHUNK = 256
NGATH = CHUNK // IDXW


def _make_sc_lookup(bl):
    info = plsc.get_sparse_core_info()
    n_workers = info.num_cores * info.num_subcores
    tpw = bl // n_workers
    assert tpw % CHUNK == 0
    n_chunks = tpw // CHUNK
    assert n_chunks % 2 == 0 and n_chunks >= 4
    rows_per_tile = NCOMB // 16

    mesh = plsc.VectorSubcoreMesh(core_axis_name="c", subcore_axis_name="s")

    def body(i1_hbm, i2_hbm, w1_hbm, w2_hbm, out_hbm,
             w1_v, w2_v, idx1_v, idx2_v, idxc_v, rows_v, sem_g, sem_s, wsum):
        cid = lax.axis_index("c")
        sid = lax.axis_index("s")
        wid = sid * info.num_cores + cid

        pltpu.sync_copy(w1_hbm, w1_v)
        pltpu.sync_copy(w2_hbm, w2_v)

        def build_row(r, carry):
            row = sid * rows_per_tile + r
            a = row // NSQ
            b = lax.rem(row, NSQ)
            for k in range(EMBED // LANES):
                v = (w1_v[pl.ds(a * EMBED + k * LANES, LANES)]
                     + w2_v[pl.ds(b * EMBED + k * LANES, LANES)])
                rows_v[0][r, pl.ds(k * LANES, LANES)] = v
            return carry

        lax.fori_loop(0, rows_per_tile, build_row, 0)
        pltpu.sync_copy(rows_v[0], wsum.at[pl.ds(sid * rows_per_tile, rows_per_tile)])
        plsc.subcore_barrier()

        base0 = wid * tpw

        def load_and_fire(g, buf):
            base = base0 + g * CHUNK
            pltpu.sync_copy(i1_hbm.at[pl.ds(base, CHUNK)], idx1_v[buf])
            pltpu.sync_copy(i2_hbm.at[pl.ds(base, CHUNK)], idx2_v[buf])
            for j in range(NGATH):
                for k in range(IDXW // LANES):
                    s = pl.ds(j * IDXW + k * LANES, LANES)
                    idxc_v[buf][j, pl.ds(k * LANES, LANES)] = (
                        idx1_v[buf][s] * NSQ + idx2_v[buf][s])
            for j in range(NGATH):
                pltpu.async_copy(
                    wsum.at[idxc_v[buf].at[j]],
                    rows_v[buf].at[pl.ds(j * IDXW, IDXW)],
                    sem_g[buf])

        def wait_gather(buf):
            pltpu.make_async_copy(
                wsum.at[idxc_v[buf].at[0]],
                rows_v[buf].at[pl.ds(0, IDXW)],
                sem_g[buf]).wait()

        def fire_store(g, buf):
            base = base0 + g * CHUNK
            pltpu.async_copy(rows_v[buf], out_hbm.at[pl.ds(base, CHUNK)],
                             sem_s[buf])

        def wait_store(buf):
            pltpu.make_async_copy(rows_v[buf], out_hbm.at[pl.ds(0, CHUNK)],
                                  sem_s[buf]).wait()

        def do_g(g, buf, first, last):
            nbuf = 1 - buf
            for _ in range(NGATH):
                wait_gather(buf)
            fire_store(g, buf)
            if not first:
                wait_store(nbuf)
            if not last:
                load_and_fire(g + 1, nbuf)

        load_and_fire(0, 0)
        do_g(0, 0, first=True, last=False)

        def step(p, carry):
            do_g(2 * p + 1, 1, first=False, last=False)
            do_g(2 * p + 2, 0, first=False, last=False)
            return carry

        lax.fori_loop(0, n_chunks // 2 - 1, step, 0)
        do_g(n_chunks - 1, 1, first=False, last=True)
        wait_store(1)

    return pl.kernel(
        body,
        out_type=jax.ShapeDtypeStruct((bl, EMBED), jnp.float32),
        mesh=mesh,
        scratch_types=[
            pltpu.VMEM((NSQ * EMBED,), jnp.float32),
            pltpu.VMEM((NSQ * EMBED,), jnp.float32),
            [pltpu.VMEM((CHUNK,), jnp.int32) for _ in range(2)],
            [pltpu.VMEM((CHUNK,), jnp.int32) for _ in range(2)],
            [pltpu.VMEM((NGATH, IDXW), jnp.int32) for _ in range(2)],
            [pltpu.VMEM((CHUNK, EMBED), jnp.float32) for _ in range(2)],
            [pltpu.SemaphoreType.DMA for _ in range(2)],
            [pltpu.SemaphoreType.DMA for _ in range(2)],
            pltpu.VMEM_SHARED((NCOMB, EMBED), jnp.float32),
        ],
    )


def kernel(initial_position_indexes, destination_indexes, W_initial, W_destination):
    b, l = initial_position_indexes.shape
    bl = b * l
    i1, i2 = lax.optimization_barrier((
        initial_position_indexes.reshape(bl).astype(jnp.int32),
        destination_indexes.reshape(bl).astype(jnp.int32),
    ))
    w1 = W_initial.reshape(NSQ * EMBED)
    w2 = W_destination.reshape(NSQ * EMBED)
    out = _make_sc_lookup(bl)(i1, i2, w1, w2)
    return out.reshape(b, l, EMBED)


# speedup vs baseline: 1.4504x; 1.4504x over previous
import jax
import jax.numpy as jnp
from jax import lax
from jax.experimental import pallas as pl
from jax.experimental.pallas import tpu as pltpu
from jax.experimental.pallas import tpu_sc as plsc

EMBED = 128
NSQ = 64
NCOMB = NSQ * NSQ
LANES = 16
IDXW = 128
CHUNK = 128
NGATH = CHUNK // IDXW
NBUF = 4


def _make_sc_lookup(bl):
    info = plsc.get_sparse_core_info()
    n_workers = info.num_cores * info.num_subcores
    tpw = bl // n_workers
    assert tpw % CHUNK == 0
    n_chunks = tpw // CHUNK
    assert n_chunks >= 4 * NBUF
    rows_per_tile = NCOMB // 16

    mesh = plsc.VectorSubcoreMesh(core_axis_name="c", subcore_axis_name="s")

    def body(i1_hbm, i2_hbm, w1_hbm, w2_hbm, out_hbm,
             w1_v, w2_v, idx1_v, idx2_v, idxc_v, rows_v,
             sem_i, sem_g, sem_s, wsum):
        cid = lax.axis_index("c")
        sid = lax.axis_index("s")
        wid = sid * info.num_cores + cid

        pltpu.sync_copy(w1_hbm, w1_v)
        pltpu.sync_copy(w2_hbm, w2_v)

        for h in range(rows_per_tile // CHUNK):

            def build_row(r, carry):
                row = sid * rows_per_tile + h * CHUNK + r
                a = row // NSQ
                b = lax.rem(row, NSQ)
                for k in range(EMBED // LANES):
                    v = (w1_v[pl.ds(a * EMBED + k * LANES, LANES)]
                         + w2_v[pl.ds(b * EMBED + k * LANES, LANES)])
                    rows_v[0][r, pl.ds(k * LANES, LANES)] = v
                return carry

            lax.fori_loop(0, CHUNK, build_row, 0)
            pltpu.sync_copy(
                rows_v[0],
                wsum.at[pl.ds(sid * rows_per_tile + h * CHUNK, CHUNK)])
        plsc.subcore_barrier()

        base0 = wid * tpw

        def fire_idx(g, s):
            base = base0 + g * CHUNK
            pltpu.async_copy(i1_hbm.at[pl.ds(base, CHUNK)], idx1_v[s], sem_i[s])
            pltpu.async_copy(i2_hbm.at[pl.ds(base, CHUNK)], idx2_v[s], sem_i[s])

        def wait_idx(s):
            pltpu.make_async_copy(i1_hbm.at[pl.ds(0, CHUNK)], idx1_v[s], sem_i[s]).wait()
            pltpu.make_async_copy(i2_hbm.at[pl.ds(0, CHUNK)], idx2_v[s], sem_i[s]).wait()

        def compute_idxc(s):
            for j in range(NGATH):
                for k in range(IDXW // LANES):
                    sl = pl.ds(j * IDXW + k * LANES, LANES)
                    idxc_v[s][j, pl.ds(k * LANES, LANES)] = (
                        idx1_v[s][sl] * NSQ + idx2_v[s][sl])

        def fire_gather(s):
            for j in range(NGATH):
                pltpu.async_copy(
                    wsum.at[idxc_v[s].at[j]],
                    rows_v[s].at[pl.ds(j * IDXW, IDXW)],
                    sem_g[s])

        def wait_gather(s):
            for _ in range(NGATH):
                pltpu.make_async_copy(
                    wsum.at[idxc_v[s].at[0]],
                    rows_v[s].at[pl.ds(0, IDXW)],
                    sem_g[s]).wait()

        def fire_store(g, s):
            base = base0 + g * CHUNK
            pltpu.async_copy(rows_v[s], out_hbm.at[pl.ds(base, CHUNK)], sem_s[s])

        def wait_store(s):
            pltpu.make_async_copy(rows_v[s], out_hbm.at[pl.ds(0, CHUNK)],
                                  sem_s[s]).wait()

        def do_chunk(g, r0, *, ws=True, fire_next=True, idx2ahead=True):
            b0 = r0 % NBUF
            b1 = (r0 + 1) % NBUF
            b2 = (r0 + 2) % NBUF
            if fire_next:
                wait_idx(b1)
                compute_idxc(b1)
                if ws:
                    wait_store(b1)
                fire_gather(b1)
            wait_gather(b0)
            fire_store(g, b0)
            if idx2ahead:
                fire_idx(g + 2, b2)

        head = NBUF - 1
        tail = (n_chunks - head) % NBUF + NBUF
        loop_n = (n_chunks - head - tail) // NBUF
        assert loop_n >= 1

        fire_idx(0, 0)
        fire_idx(1, 1)
        wait_idx(0)
        compute_idxc(0)
        fire_gather(0)
        for g in range(head):
            do_chunk(g, g % NBUF, ws=False)

        def step(p, carry):
            g = NBUF * p + head
            for r in range(NBUF):
                do_chunk(g + r, (head + r) % NBUF)
            return carry

        lax.fori_loop(0, loop_n, step, 0)
        for t in range(tail):
            g = n_chunks - tail + t
            do_chunk(g, g % NBUF,
                     fire_next=(g + 1 <= n_chunks - 1),
                     idx2ahead=(g + 2 <= n_chunks - 1))
        for s in range(NBUF):
            wait_store(s)

    return pl.kernel(
        body,
        out_type=jax.ShapeDtypeStruct((bl, EMBED), jnp.float32),
        mesh=mesh,
        scratch_types=[
            pltpu.VMEM((NSQ * EMBED,), jnp.float32),
            pltpu.VMEM((NSQ * EMBED,), jnp.float32),
            [pltpu.VMEM((CHUNK,), jnp.int32) for _ in range(NBUF)],
            [pltpu.VMEM((CHUNK,), jnp.int32) for _ in range(NBUF)],
            [pltpu.VMEM((NGATH, IDXW), jnp.int32) for _ in range(NBUF)],
            [pltpu.VMEM((CHUNK, EMBED), jnp.float32) for _ in range(NBUF)],
            [pltpu.SemaphoreType.DMA for _ in range(NBUF)],
            [pltpu.SemaphoreType.DMA for _ in range(NBUF)],
            [pltpu.SemaphoreType.DMA for _ in range(NBUF)],
            pltpu.VMEM_SHARED((NCOMB, EMBED), jnp.float32),
        ],
    )


def kernel(initial_position_indexes, destination_indexes, W_initial, W_destination):
    b, l = initial_position_indexes.shape
    bl = b * l
    i1 = initial_position_indexes.reshape(bl).astype(jnp.int32)
    i2 = destination_indexes.reshape(bl).astype(jnp.int32)
    w1 = W_initial.reshape(NSQ * EMBED)
    w2 = W_destination.reshape(NSQ * EMBED)
    out = _make_sc_lookup(bl)(i1, i2, w1, w2)
    return out.reshape(b, l, EMBED)


# speedup vs baseline: 1.4510x; 1.0005x over previous
import jax
import jax.numpy as jnp
from jax import lax
from jax.experimental import pallas as pl
from jax.experimental.pallas import tpu as pltpu
from jax.experimental.pallas import tpu_sc as plsc

EMBED = 128
NSQ = 64
NCOMB = NSQ * NSQ
LANES = 16
IDXW = 128
CHUNK = 128
NGATH = CHUNK // IDXW
NBUF = 5


def _make_sc_lookup(bl):
    info = plsc.get_sparse_core_info()
    n_workers = info.num_cores * info.num_subcores
    tpw = bl // n_workers
    assert tpw % CHUNK == 0
    n_chunks = tpw // CHUNK
    assert n_chunks >= 4 * NBUF
    rows_per_tile = NCOMB // 16

    mesh = plsc.VectorSubcoreMesh(core_axis_name="c", subcore_axis_name="s")

    def body(i1_hbm, i2_hbm, w1_hbm, w2_hbm, out_hbm,
             idx1_v, idx2_v, idxc_v, rows_v,
             sem_i, sem_g, sem_s, wsum):
        cid = lax.axis_index("c")
        sid = lax.axis_index("s")
        wid = sid * info.num_cores + cid

        pltpu.sync_copy(w1_hbm, rows_v[0].at[pl.ds(0, NSQ)])
        pltpu.sync_copy(w2_hbm, rows_v[0].at[pl.ds(NSQ, NSQ)])

        for h in range(rows_per_tile // CHUNK):

            def build_row(r, carry):
                row = sid * rows_per_tile + h * CHUNK + r
                a = row // NSQ
                b = lax.rem(row, NSQ)
                for k in range(EMBED // LANES):
                    v = (rows_v[0][a, pl.ds(k * LANES, LANES)]
                         + rows_v[0][NSQ + b, pl.ds(k * LANES, LANES)])
                    rows_v[1][r, pl.ds(k * LANES, LANES)] = v
                return carry

            lax.fori_loop(0, CHUNK, build_row, 0)
            pltpu.sync_copy(
                rows_v[1],
                wsum.at[pl.ds(sid * rows_per_tile + h * CHUNK, CHUNK)])
        plsc.subcore_barrier()

        base0 = wid * tpw

        def fire_idx(g, s):
            base = base0 + g * CHUNK
            pltpu.async_copy(i1_hbm.at[pl.ds(base, CHUNK)], idx1_v[s], sem_i[s])
            pltpu.async_copy(i2_hbm.at[pl.ds(base, CHUNK)], idx2_v[s], sem_i[s])

        def wait_idx(s):
            pltpu.make_async_copy(i1_hbm.at[pl.ds(0, CHUNK)], idx1_v[s], sem_i[s]).wait()
            pltpu.make_async_copy(i2_hbm.at[pl.ds(0, CHUNK)], idx2_v[s], sem_i[s]).wait()

        def compute_idxc(s):
            for j in range(NGATH):
                for k in range(IDXW // LANES):
                    sl = pl.ds(j * IDXW + k * LANES, LANES)
                    idxc_v[s][j, pl.ds(k * LANES, LANES)] = (
                        idx1_v[s][sl] * NSQ + idx2_v[s][sl])

        def fire_gather(s):
            for j in range(NGATH):
                pltpu.async_copy(
                    wsum.at[idxc_v[s].at[j]],
                    rows_v[s].at[pl.ds(j * IDXW, IDXW)],
                    sem_g[s])

        def wait_gather(s):
            for _ in range(NGATH):
                pltpu.make_async_copy(
                    wsum.at[idxc_v[s].at[0]],
                    rows_v[s].at[pl.ds(0, IDXW)],
                    sem_g[s]).wait()

        def fire_store(g, s):
            base = base0 + g * CHUNK
            pltpu.async_copy(rows_v[s], out_hbm.at[pl.ds(base, CHUNK)], sem_s[s])

        def wait_store(s):
            pltpu.make_async_copy(rows_v[s], out_hbm.at[pl.ds(0, CHUNK)],
                                  sem_s[s]).wait()

        def do_chunk(g, r0, *, ws=True, fire_next=True, idx2ahead=True):
            b0 = r0 % NBUF
            b1 = (r0 + 1) % NBUF
            b2 = (r0 + 2) % NBUF
            if fire_next:
                wait_idx(b1)
                compute_idxc(b1)
                if ws:
                    wait_store(b1)
                fire_gather(b1)
            wait_gather(b0)
            fire_store(g, b0)
            if idx2ahead:
                fire_idx(g + 2, b2)

        head = NBUF - 1
        tail = (n_chunks - head) % NBUF + NBUF
        loop_n = (n_chunks - head - tail) // NBUF
        assert loop_n >= 1

        fire_idx(0, 0)
        fire_idx(1, 1)
        wait_idx(0)
        compute_idxc(0)
        fire_gather(0)
        for g in range(head):
            do_chunk(g, g % NBUF, ws=False)

        def step(p, carry):
            g = NBUF * p + head
            for r in range(NBUF):
                do_chunk(g + r, (head + r) % NBUF)
            return carry

        lax.fori_loop(0, loop_n, step, 0)
        for t in range(tail):
            g = n_chunks - tail + t
            do_chunk(g, g % NBUF,
                     fire_next=(g + 1 <= n_chunks - 1),
                     idx2ahead=(g + 2 <= n_chunks - 1))
        for s in range(NBUF):
            wait_store(s)

    return pl.kernel(
        body,
        out_type=jax.ShapeDtypeStruct((bl, EMBED), jnp.float32),
        mesh=mesh,
        scratch_types=[
            [pltpu.VMEM((CHUNK,), jnp.int32) for _ in range(NBUF)],
            [pltpu.VMEM((CHUNK,), jnp.int32) for _ in range(NBUF)],
            [pltpu.VMEM((NGATH, IDXW), jnp.int32) for _ in range(NBUF)],
            [pltpu.VMEM((CHUNK, EMBED), jnp.float32) for _ in range(NBUF)],
            [pltpu.SemaphoreType.DMA for _ in range(NBUF)],
            [pltpu.SemaphoreType.DMA for _ in range(NBUF)],
            [pltpu.SemaphoreType.DMA for _ in range(NBUF)],
            pltpu.VMEM_SHARED((NCOMB, EMBED), jnp.float32),
        ],
    )


def kernel(initial_position_indexes, destination_indexes, W_initial, W_destination):
    b, l = initial_position_indexes.shape
    bl = b * l
    i1 = initial_position_indexes.reshape(bl).astype(jnp.int32)
    i2 = destination_indexes.reshape(bl).astype(jnp.int32)
    out = _make_sc_lookup(bl)(i1, i2, W_initial, W_destination)
    return out.reshape(b, l, EMBED)


# speedup vs baseline: 1.5421x; 1.0627x over previous
import jax
import jax.numpy as jnp
from jax import lax
from jax.experimental import pallas as pl
from jax.experimental.pallas import tpu as pltpu
from jax.experimental.pallas import tpu_sc as plsc

EMBED = 128
NSQ = 64
NCOMB = NSQ * NSQ
LANES = 16
CHUNK = 128
NBUF = 5


def _make_sc_lookup(bl):
    info = plsc.get_sparse_core_info()
    n_workers = info.num_cores * info.num_subcores
    tpw = bl // n_workers
    assert tpw % CHUNK == 0
    n_chunks = tpw // CHUNK
    assert n_chunks >= 4 * NBUF
    rows_per_tile = NCOMB // 16

    mesh = plsc.VectorSubcoreMesh(core_axis_name="c", subcore_axis_name="s")

    def body(idx_hbm, w1_hbm, w2_hbm, out_hbm,
             idxc_v, rows_v, sem_i, sem_g, sem_s, wsum):
        cid = lax.axis_index("c")
        sid = lax.axis_index("s")
        wid = sid * info.num_cores + cid

        pltpu.sync_copy(w1_hbm, rows_v[0].at[pl.ds(0, NSQ)])
        pltpu.sync_copy(w2_hbm, rows_v[0].at[pl.ds(NSQ, NSQ)])

        for h in range(rows_per_tile // CHUNK):

            def build_row(r, carry):
                row = sid * rows_per_tile + h * CHUNK + r
                a = row // NSQ
                b = lax.rem(row, NSQ)
                for k in range(EMBED // LANES):
                    v = (rows_v[0][a, pl.ds(k * LANES, LANES)]
                         + rows_v[0][NSQ + b, pl.ds(k * LANES, LANES)])
                    rows_v[1][r, pl.ds(k * LANES, LANES)] = v
                return carry

            lax.fori_loop(0, CHUNK, build_row, 0)
            pltpu.sync_copy(
                rows_v[1],
                wsum.at[pl.ds(sid * rows_per_tile + h * CHUNK, CHUNK)])
        plsc.subcore_barrier()

        base0 = wid * tpw

        def fire_idx(g, s):
            base = base0 + g * CHUNK
            pltpu.async_copy(idx_hbm.at[pl.ds(base, CHUNK)], idxc_v[s], sem_i[s])

        def wait_idx(s):
            pltpu.make_async_copy(idx_hbm.at[pl.ds(0, CHUNK)], idxc_v[s],
                                  sem_i[s]).wait()

        def fire_gather(s):
            pltpu.async_copy(wsum.at[idxc_v[s]], rows_v[s], sem_g[s])

        def wait_gather(s):
            pltpu.make_async_copy(wsum.at[idxc_v[s]], rows_v[s],
                                  sem_g[s]).wait()

        def fire_store(g, s):
            base = base0 + g * CHUNK
            pltpu.async_copy(rows_v[s], out_hbm.at[pl.ds(base, CHUNK)], sem_s[s])

        def wait_store(s):
            pltpu.make_async_copy(rows_v[s], out_hbm.at[pl.ds(0, CHUNK)],
                                  sem_s[s]).wait()

        def do_chunk(g, r0, *, ws=True, fire_next=True, idx2ahead=True):
            b0 = r0 % NBUF
            b1 = (r0 + 1) % NBUF
            b2 = (r0 + 2) % NBUF
            if fire_next:
                wait_idx(b1)
                if ws:
                    wait_store(b1)
                fire_gather(b1)
            wait_gather(b0)
            fire_store(g, b0)
            if idx2ahead:
                fire_idx(g + 2, b2)

        head = NBUF - 1
        tail = (n_chunks - head) % NBUF + NBUF
        loop_n = (n_chunks - head - tail) // NBUF
        assert loop_n >= 1

        fire_idx(0, 0)
        fire_idx(1, 1)
        wait_idx(0)
        fire_gather(0)
        for g in range(head):
            do_chunk(g, g % NBUF, ws=False)

        def step(p, carry):
            g = NBUF * p + head
            for r in range(NBUF):
                do_chunk(g + r, (head + r) % NBUF)
            return carry

        lax.fori_loop(0, loop_n, step, 0)
        for t in range(tail):
            g = n_chunks - tail + t
            do_chunk(g, g % NBUF,
                     fire_next=(g + 1 <= n_chunks - 1),
                     idx2ahead=(g + 2 <= n_chunks - 1))
        for s in range(NBUF):
            wait_store(s)

    return pl.kernel(
        body,
        out_type=jax.ShapeDtypeStruct((bl, EMBED), jnp.float32),
        mesh=mesh,
        scratch_types=[
            [pltpu.VMEM((CHUNK,), jnp.int32) for _ in range(NBUF)],
            [pltpu.VMEM((CHUNK, EMBED), jnp.float32) for _ in range(NBUF)],
            [pltpu.SemaphoreType.DMA for _ in range(NBUF)],
            [pltpu.SemaphoreType.DMA for _ in range(NBUF)],
            [pltpu.SemaphoreType.DMA for _ in range(NBUF)],
            pltpu.VMEM_SHARED((NCOMB, EMBED), jnp.float32),
        ],
    )


def kernel(initial_position_indexes, destination_indexes, W_initial, W_destination):
    b, l = initial_position_indexes.shape
    bl = b * l
    idx = (initial_position_indexes.astype(jnp.int32) * NSQ
           + destination_indexes.astype(jnp.int32)).reshape(bl)
    out = _make_sc_lookup(bl)(idx, W_initial, W_destination)
    return out.reshape(b, l, EMBED)
